# baseline (speedup 1.0000x reference)
import dataclasses

import jax
import jax.numpy as jnp
from jax import lax
from jax.experimental import pallas as pl
from jax.experimental.pallas import tpu as pltpu
from jax.experimental.pallas import tpu_sc as plsc

B = 4096
D = 128
R = 100
RP = 128
NC = 2
NS = 16
LANES = 16
NW = NC * NS
BPW = B // NW
BC = 4096
GROUPS = BPW // LANES
MARGIN = 1.0
BIG = 1e30


def _tc_dist2_body(v_ref, x_ref, o_ref):
    v = v_ref[...]
    x = x_ref[...]
    n2 = jnp.sum(x * x, axis=1, keepdims=True)
    u = x / jnp.maximum(jnp.sqrt(n2), 1e-12)
    un2 = jnp.sum(u * u, axis=1)
    dots = lax.dot_general(
        v, u, (((1,), (1,)), ((), ())),
        preferred_element_type=jnp.float32,
        precision=lax.Precision.HIGHEST)
    v2 = jnp.sum(v * v, axis=1, keepdims=True)
    d2 = jnp.maximum(v2 + un2[None, :] - 2.0 * dots, 0.0)
    r_ids = lax.broadcasted_iota(jnp.int32, (RP, 1), 0)
    o_ref[...] = jnp.where(r_ids >= R, BIG, d2)


def _nsqrt(x):
    i = lax.bitcast_convert_type(x, jnp.int32)
    y = lax.bitcast_convert_type(
        jnp.int32(0x5F3759DF) - (i >> 1), jnp.float32)
    for _ in range(3):
        y = y * (1.5 - 0.5 * x * y * y)
    return x * y


def _sc_body(d2_hbm, y_hbm, out_hbm, d2_v, y_v, acc_v, dsem, ysem):
    cid = lax.axis_index("c")
    sid = lax.axis_index("s")
    wid = sid * NC + cid
    base = pl.multiple_of(wid * BPW, BPW)
    dc = pltpu.make_async_copy(d2_hbm.at[:, pl.ds(base, BPW)], d2_v, dsem)
    dc.start()
    yc = pltpu.make_async_copy(y_hbm.at[pl.ds(base, BPW)], y_v, ysem)
    yc.start()
    yc.wait()
    dc.wait()

    lane = jnp.arange(LANES, dtype=jnp.int32)
    acc_v[...] = jnp.zeros((LANES,), jnp.float32)

    @pl.loop(0, GROUPS)
    def _(g):
        off = pl.multiple_of(g * LANES, LANES)
        yv = y_v[pl.ds(off, LANES)]
        ib = lane + off
        yd2 = plsc.load_gather(d2_v, [yv, ib])
        plsc.store_scatter(d2_v, [yv, ib],
                           jnp.full((LANES,), BIG, jnp.float32))
        accs = [jnp.full((LANES,), BIG, jnp.float32) for _ in range(8)]
        for r in range(R):
            val = d2_v[r, pl.ds(off, LANES)]
            accs[r % 8] = jnp.minimum(accs[r % 8], val)
        while len(accs) > 1:
            accs = [jnp.minimum(a, b) for a, b in zip(accs[::2], accs[1::2])]
        m2 = accs[0]
        sy = _nsqrt(yd2)
        sm = _nsqrt(m2)
        t = jnp.minimum(sm, sy + 10000.0)
        acc_v[...] = acc_v[...] + (MARGIN + sy - t) * (1.0 / B)

    pltpu.sync_copy(acc_v, out_hbm.at[wid])


def kernel(wo, rel_weight, in_y):
    x2d = wo.reshape(B, D)
    vpad = jnp.zeros((RP, D), jnp.float32).at[:R].set(rel_weight)
    y = in_y.reshape(B).astype(jnp.int32)

    d2t = pl.pallas_call(
        _tc_dist2_body,
        grid=(B // BC,),
        in_specs=[
            pl.BlockSpec((RP, D), lambda i: (0, 0)),
            pl.BlockSpec((BC, D), lambda i: (i, 0)),
        ],
        out_specs=pl.BlockSpec((RP, BC), lambda i: (0, i)),
        out_shape=jax.ShapeDtypeStruct((RP, B), jnp.float32),
    )(vpad, x2d)

    cp = pltpu.CompilerParams()
    if "needs_layout_passes" in pltpu.CompilerParams.__dataclass_fields__:
        cp = dataclasses.replace(cp, needs_layout_passes=False)
    sc_stage = pl.kernel(
        _sc_body,
        out_type=jax.ShapeDtypeStruct((NW, LANES), jnp.float32),
        mesh=plsc.VectorSubcoreMesh(core_axis_name="c", subcore_axis_name="s"),
        compiler_params=cp,
        scratch_types=[
            pltpu.VMEM((RP, BPW), jnp.float32),
            pltpu.VMEM((BPW,), jnp.int32),
            pltpu.VMEM((LANES,), jnp.float32),
            pltpu.SemaphoreType.DMA,
            pltpu.SemaphoreType.DMA,
        ],
    )
    partials = sc_stage(d2t, y)
    return jnp.sum(partials)


# speedup vs baseline: 1.0423x; 1.0423x over previous
import dataclasses

import jax
import jax.numpy as jnp
from jax import lax
from jax.experimental import pallas as pl
from jax.experimental.pallas import tpu as pltpu
from jax.experimental.pallas import tpu_sc as plsc

B = 4096
D = 128
R = 100
RP = 128
NC = 2
NS = 16
LANES = 16
NW = NC * NS
BPW = B // NW
BC = 2048
GROUPS = BPW // LANES
MARGIN = 1.0
BIG = 1e30


def _tc_dist2_body(v_ref, x_ref, o_ref):
    v = v_ref[...]
    x = x_ref[...]
    n2 = jnp.sum(x * x, axis=1, keepdims=True)
    u = x / jnp.maximum(jnp.sqrt(n2), 1e-12)
    un2 = jnp.sum(u * u, axis=1)
    dims = (((1,), (1,)), ((), ()))
    uh = u.astype(jnp.bfloat16)
    ul = (u - uh.astype(jnp.float32)).astype(jnp.bfloat16)
    vh = v.astype(jnp.bfloat16)
    vl = (v - vh.astype(jnp.float32)).astype(jnp.bfloat16)
    f32 = jnp.float32
    dots = (lax.dot_general(vh, uh, dims, preferred_element_type=f32)
            + lax.dot_general(vh, ul, dims, preferred_element_type=f32)
            + lax.dot_general(vl, uh, dims, preferred_element_type=f32))
    v2 = jnp.sum(v * v, axis=1, keepdims=True)
    d2 = jnp.maximum(v2 + un2[None, :] - 2.0 * dots, 0.0)
    r_ids = lax.broadcasted_iota(jnp.int32, (RP, 1), 0)
    o_ref[...] = jnp.where(r_ids >= R, BIG, d2)


def _nsqrt(x):
    i = lax.bitcast_convert_type(x, jnp.int32)
    y = lax.bitcast_convert_type(
        jnp.int32(0x5F3759DF) - (i >> 1), jnp.float32)
    for _ in range(3):
        y = y * (1.5 - 0.5 * x * y * y)
    return x * y


def _sc_body(d2_hbm, y_hbm, out_hbm, d2_v, y_v, acc_v, dsem, ysem):
    cid = lax.axis_index("c")
    sid = lax.axis_index("s")
    wid = sid * NC + cid
    base = pl.multiple_of(wid * BPW, BPW)
    dc = pltpu.make_async_copy(d2_hbm.at[:, pl.ds(base, BPW)], d2_v, dsem)
    dc.start()
    yc = pltpu.make_async_copy(y_hbm.at[pl.ds(base, BPW)], y_v, ysem)
    yc.start()
    yc.wait()
    dc.wait()

    lane = jnp.arange(LANES, dtype=jnp.int32)
    acc_v[...] = jnp.zeros((LANES,), jnp.float32)

    @pl.loop(0, GROUPS)
    def _(g):
        off = pl.multiple_of(g * LANES, LANES)
        yv = y_v[pl.ds(off, LANES)]
        ib = lane + off
        yd2 = plsc.load_gather(d2_v, [yv, ib])
        plsc.store_scatter(d2_v, [yv, ib],
                           jnp.full((LANES,), BIG, jnp.float32))
        accs = [jnp.full((LANES,), BIG, jnp.float32) for _ in range(8)]
        for r in range(R):
            val = d2_v[r, pl.ds(off, LANES)]
            accs[r % 8] = jnp.minimum(accs[r % 8], val)
        while len(accs) > 1:
            accs = [jnp.minimum(a, b) for a, b in zip(accs[::2], accs[1::2])]
        m2 = accs[0]
        sy = _nsqrt(yd2)
        sm = _nsqrt(m2)
        t = jnp.minimum(sm, sy + 10000.0)
        acc_v[...] = acc_v[...] + (MARGIN + sy - t) * (1.0 / B)

    pltpu.sync_copy(acc_v, out_hbm.at[wid])


def kernel(wo, rel_weight, in_y):
    x2d = wo.reshape(B, D)
    vpad = jnp.zeros((RP, D), jnp.float32).at[:R].set(rel_weight)
    y = in_y.reshape(B).astype(jnp.int32)

    d2t = pl.pallas_call(
        _tc_dist2_body,
        grid=(B // BC,),
        in_specs=[
            pl.BlockSpec((RP, D), lambda i: (0, 0)),
            pl.BlockSpec((BC, D), lambda i: (i, 0)),
        ],
        out_specs=pl.BlockSpec((RP, BC), lambda i: (0, i)),
        out_shape=jax.ShapeDtypeStruct((RP, B), jnp.float32),
    )(vpad, x2d)

    cp = pltpu.CompilerParams()
    if "needs_layout_passes" in pltpu.CompilerParams.__dataclass_fields__:
        cp = dataclasses.replace(cp, needs_layout_passes=False)
    sc_stage = pl.kernel(
        _sc_body,
        out_type=jax.ShapeDtypeStruct((NW, LANES), jnp.float32),
        mesh=plsc.VectorSubcoreMesh(core_axis_name="c", subcore_axis_name="s"),
        compiler_params=cp,
        scratch_types=[
            pltpu.VMEM((RP, BPW), jnp.float32),
            pltpu.VMEM((BPW,), jnp.int32),
            pltpu.VMEM((LANES,), jnp.float32),
            pltpu.SemaphoreType.DMA,
            pltpu.SemaphoreType.DMA,
        ],
    )
    partials = sc_stage(d2t, y)
    return jnp.sum(partials)


# speedup vs baseline: 1.0436x; 1.0013x over previous
import dataclasses

import jax
import jax.numpy as jnp
from jax import lax
from jax.experimental import pallas as pl
from jax.experimental.pallas import tpu as pltpu
from jax.experimental.pallas import tpu_sc as plsc

B = 4096
D = 128
R = 100
RP = 128
NC = 2
NS = 16
LANES = 16
NW = NC * NS
BPW = B // NW
BC = 2048
GROUPS = BPW // LANES
MARGIN = 1.0
BIG = 1e30


def _tc_dist2_body(v_ref, x_ref, o_ref):
    v = v_ref[...]
    x = x_ref[...]
    n2 = jnp.sum(x * x, axis=1, keepdims=True)
    u = x / jnp.maximum(jnp.sqrt(n2), 1e-12)
    un2 = jnp.sum(u * u, axis=1)
    dims = (((1,), (1,)), ((), ()))
    uh = u.astype(jnp.bfloat16)
    ul = (u - uh.astype(jnp.float32)).astype(jnp.bfloat16)
    vh = v.astype(jnp.bfloat16)
    vl = (v - vh.astype(jnp.float32)).astype(jnp.bfloat16)
    f32 = jnp.float32
    dots = (lax.dot_general(vh, uh, dims, preferred_element_type=f32)
            + lax.dot_general(vh, ul, dims, preferred_element_type=f32)
            + lax.dot_general(vl, uh, dims, preferred_element_type=f32))
    v2 = jnp.sum(v * v, axis=1, keepdims=True)
    d2 = jnp.maximum(v2 + un2[None, :] - 2.0 * dots, 0.0)
    r_ids = lax.broadcasted_iota(jnp.int32, (RP, 1), 0)
    o_ref[...] = jnp.where(r_ids >= R, BIG, d2)


def _nsqrt(x):
    i = lax.bitcast_convert_type(x, jnp.int32)
    y = lax.bitcast_convert_type(
        jnp.int32(0x5F3759DF) - (i >> 1), jnp.float32)
    for _ in range(3):
        y = y * (1.5 - 0.5 * x * y * y)
    return x * y


def _sc_body(d2_hbm, y_hbm, out_hbm, d2_v, y_v, acc_v, dsem, ysem):
    cid = lax.axis_index("c")
    sid = lax.axis_index("s")
    wid = sid * NC + cid
    base = pl.multiple_of(wid * BPW, BPW)
    dc = pltpu.make_async_copy(d2_hbm.at[:, pl.ds(base, BPW)], d2_v, dsem)
    dc.start()
    yc = pltpu.make_async_copy(y_hbm.at[pl.ds(base, BPW)], y_v, ysem)
    yc.start()
    yc.wait()
    dc.wait()

    lane = jnp.arange(LANES, dtype=jnp.int32)
    acc_v[...] = jnp.zeros((LANES,), jnp.float32)

    @pl.loop(0, GROUPS)
    def _group(g):
        off = pl.multiple_of(g * LANES, LANES)
        yv = y_v[pl.ds(off, LANES)]
        ib = lane + off
        yd2 = plsc.load_gather(d2_v, [yv, ib])
        plsc.store_scatter(d2_v, [yv, ib],
                           jnp.full((LANES,), BIG, jnp.float32))
        accs = [jnp.full((LANES,), BIG, jnp.float32) for _ in range(8)]
        for r in range(R):
            val = d2_v[r, pl.ds(off, LANES)]
            accs[r % 8] = jnp.minimum(accs[r % 8], val)
        while len(accs) > 1:
            accs = [jnp.minimum(a, b) for a, b in zip(accs[::2], accs[1::2])]
        m2 = accs[0]
        sy = _nsqrt(yd2)
        sm = _nsqrt(m2)
        t = jnp.minimum(sm, sy + 10000.0)
        acc_v[...] = acc_v[...] + (MARGIN + sy - t) * (1.0 / B)

    pltpu.sync_copy(acc_v, out_hbm.at[wid])


def kernel(wo, rel_weight, in_y):
    x2d = wo.reshape(B, D)
    vpad = jnp.zeros((RP, D), jnp.float32).at[:R].set(rel_weight)
    y = in_y.reshape(B).astype(jnp.int32)

    d2t = pl.pallas_call(
        _tc_dist2_body,
        grid=(B // BC,),
        in_specs=[
            pl.BlockSpec((RP, D), lambda i: (0, 0)),
            pl.BlockSpec((BC, D), lambda i: (i, 0)),
        ],
        out_specs=pl.BlockSpec((RP, BC), lambda i: (0, i)),
        out_shape=jax.ShapeDtypeStruct((RP, B), jnp.float32),
    )(vpad, x2d)

    cp = pltpu.CompilerParams()
    if "needs_layout_passes" in pltpu.CompilerParams.__dataclass_fields__:
        cp = dataclasses.replace(cp, needs_layout_passes=False)
    sc_stage = pl.kernel(
        _sc_body,
        out_type=jax.ShapeDtypeStruct((NW, LANES), jnp.float32),
        mesh=plsc.VectorSubcoreMesh(core_axis_name="c", subcore_axis_name="s"),
        compiler_params=cp,
        scratch_types=[
            pltpu.VMEM((RP, BPW), jnp.float32),
            pltpu.VMEM((BPW,), jnp.int32),
            pltpu.VMEM((LANES,), jnp.float32),
            pltpu.SemaphoreType.DMA,
            pltpu.SemaphoreType.DMA,
        ],
    )
    partials = sc_stage(d2t, y)
    return jnp.sum(partials)
